# Initial kernel scaffold; baseline (speedup 1.0000x reference)
import jax
import jax.numpy as jnp
from jax.experimental import pallas as pl


def kernel(x, edge_index, W, b):
    raise NotImplementedError("write your pallas kernel here")


# speedup vs baseline: 2.3026x; 2.3026x over previous
import functools

import jax
import jax.numpy as jnp
from jax import lax
from jax.experimental import pallas as pl
from jax.experimental.pallas import tpu as pltpu
from jax.experimental.pallas import tpu_sc as plsc

N = 10000
D = 768
E = 100000

NP = 10240
EP = 102400
EDGE_PAD_DST = 10000
EDGE_PAD_SRC = 10200

SEG = 6400
ROWS_PER_TILE = NP // 32
NPASS = 4
PROWS = ROWS_PER_TILE // NPASS
KE = 32
ZROW = NP - PROWS

_MESH = plsc.VectorSubcoreMesh(
    core_axis_name="c", subcore_axis_name="s", num_cores=2, num_subcores=16
)


def _deg_body(dst_hbm, cnt_hbm, dbuf, counts):
    core = lax.axis_index("c")
    sub = lax.axis_index("s")
    wid = core * 16 + sub
    per = EP // 32
    pltpu.sync_copy(dst_hbm.at[pl.ds(wid * per, per)], dbuf)

    def zero(i, _):
        counts[pl.ds(i * 16, 16)] = jnp.zeros((16,), jnp.int32)
        return 0

    lax.fori_loop(0, NP // 16, zero, 0)

    def count(i, _):
        d16 = dbuf[pl.ds(i * 16, 16)]
        plsc.addupdate_scatter(counts, [d16], jnp.ones((16,), jnp.int32))
        return 0

    lax.fori_loop(0, per // 16, count, 0)
    pltpu.sync_copy(counts, cnt_hbm.at[wid])


def _deg_counts(dstp):
    k = pl.kernel(
        _deg_body,
        out_type=jax.ShapeDtypeStruct((32, NP), jnp.int32),
        mesh=_MESH,
        scratch_types=[
            pltpu.VMEM((EP // 32,), jnp.int32),
            pltpu.VMEM((NP,), jnp.int32),
        ],
        compiler_params=pltpu.CompilerParams(needs_layout_passes=False),
    )
    return k(dstp)


def _mm_body(x_ref, w_ref, cnt_ref, g_ref, dinv_ref):
    deg = (jnp.sum(cnt_ref[...], axis=0) + 1).astype(jnp.float32)
    dinv = lax.rsqrt(deg)
    h = jnp.dot(x_ref[...], w_ref[...], preferred_element_type=jnp.float32)
    g_ref[...] = h * dinv[:, None]
    dinv_ref[...] = jnp.broadcast_to(dinv[None, :], (8, 512))


def _matmul_scale(x_p, W, counts):
    grid = NP // 512
    return pl.pallas_call(
        _mm_body,
        grid=(grid,),
        in_specs=[
            pl.BlockSpec((512, D), lambda i: (i, 0)),
            pl.BlockSpec((D, D), lambda i: (0, 0)),
            pl.BlockSpec((32, 512), lambda i: (0, i)),
        ],
        out_specs=[
            pl.BlockSpec((512, D), lambda i: (i, 0)),
            pl.BlockSpec((8, 512), lambda i: (0, i)),
        ],
        out_shape=[
            jax.ShapeDtypeStruct((NP, D), jnp.float32),
            jax.ShapeDtypeStruct((8, NP), jnp.float32),
        ],
    )(x_p, W, counts)


def _scatter_body(g_hbm, src_hbm, dst_hbm, s_hbm,
                  srcbuf, dstbuf, hitsrc, hitdst, rows, acc, sem):
    core = lax.axis_index("c")
    sub = lax.axis_index("s")
    wid = core * 16 + sub

    for p in range(NPASS):
        lo = wid * ROWS_PER_TILE + p * PROWS
        hi = lo + PROWS
        pltpu.sync_copy(g_hbm.at[pl.ds(ZROW, PROWS)], acc)

        def seg_body(s, _):
            base = s * SEG
            pltpu.sync_copy(src_hbm.at[pl.ds(base, SEG)], srcbuf)
            pltpu.sync_copy(dst_hbm.at[pl.ds(base, SEG)], dstbuf)

            def scan(i, cnt):
                d16 = dstbuf[pl.ds(i * 16, 16)]
                m = (d16 >= lo) & (d16 < hi)
                plsc.store_compressed(hitdst.at[pl.ds(cnt, 16)], d16 - lo,
                                      mask=m)
                s16 = srcbuf[pl.ds(i * 16, 16)]
                plsc.store_compressed(hitsrc.at[pl.ds(cnt, 16)], s16, mask=m)
                return cnt + jnp.sum(m.astype(jnp.int32))

            cnt = lax.fori_loop(0, SEG // 16, scan, jnp.int32(0))

            for j in range(KE // 16):
                hitdst[pl.ds(cnt + j * 16, 16)] = jnp.zeros((16,), jnp.int32)
                hitsrc[pl.ds(cnt + j * 16, 16)] = jnp.full(
                    (16,), EDGE_PAD_SRC, jnp.int32)

            def drain_hits(t, _):
                idx = hitsrc.at[pl.ds(t * KE, KE)]
                pltpu.async_copy(g_hbm.at[idx], rows, sem).wait()

                def acc_row(j, _):
                    d = hitdst[pl.ds(t * KE + j, 16)][0]
                    for c in range(D // 16):
                        plsc.addupdate(acc.at[d, pl.ds(c * 16, 16)],
                                       rows[j, pl.ds(c * 16, 16)])
                    return 0

                lax.fori_loop(0, KE, acc_row, 0)
                return 0

            lax.fori_loop(0, (cnt + KE - 1) // KE, drain_hits, 0)
            return 0

        lax.fori_loop(0, EP // SEG, seg_body, 0)
        pltpu.sync_copy(acc, s_hbm.at[pl.ds(lo, PROWS)])


def _scatter(g, srcp, dstp):
    k = pl.kernel(
        _scatter_body,
        out_type=jax.ShapeDtypeStruct((NP, D), jnp.float32),
        mesh=_MESH,
        scratch_types=[
            pltpu.VMEM((SEG,), jnp.int32),
            pltpu.VMEM((SEG,), jnp.int32),
            pltpu.VMEM((SEG + KE + 16,), jnp.int32),
            pltpu.VMEM((SEG + KE + 16,), jnp.int32),
            pltpu.VMEM((KE, D), jnp.float32),
            pltpu.VMEM((PROWS, D), jnp.float32),
            pltpu.SemaphoreType.DMA,
        ],
        compiler_params=pltpu.CompilerParams(needs_layout_passes=False),
    )
    return k(g, srcp, dstp)


def _comb_body(s_ref, g_ref, dinv_ref, b_ref, o_ref):
    dv = dinv_ref[0, :]
    o_ref[...] = (s_ref[...] + g_ref[...]) * dv[:, None] + b_ref[...]


def _combine(S, g, dinv, b2):
    grid = NP // 512
    return pl.pallas_call(
        _comb_body,
        grid=(grid,),
        in_specs=[
            pl.BlockSpec((512, D), lambda i: (i, 0)),
            pl.BlockSpec((512, D), lambda i: (i, 0)),
            pl.BlockSpec((8, 512), lambda i: (0, i)),
            pl.BlockSpec((1, D), lambda i: (0, 0)),
        ],
        out_specs=pl.BlockSpec((512, D), lambda i: (i, 0)),
        out_shape=jax.ShapeDtypeStruct((NP, D), jnp.float32),
    )(S, g, dinv, b2)


def kernel(x, edge_index, W, b):
    src = edge_index[0].astype(jnp.int32)
    dst = edge_index[1].astype(jnp.int32)
    srcp = jnp.concatenate([src, jnp.full((EP - E,), EDGE_PAD_SRC, jnp.int32)])
    dstp = jnp.concatenate([dst, jnp.full((EP - E,), EDGE_PAD_DST, jnp.int32)])
    x_p = jnp.pad(x, ((0, NP - N), (0, 0)))

    counts = _deg_counts(dstp)
    g, dinv = _matmul_scale(x_p, W, counts)
    S = _scatter(g, srcp, dstp)
    out = _combine(S, g, dinv, jnp.reshape(b, (1, D)))
    return out[:N]
